# Initial kernel scaffold; baseline (speedup 1.0000x reference)
import jax
import jax.numpy as jnp
from jax.experimental import pallas as pl


def kernel(x, edge_index, adj, W_fc, b_fc, W1, b1, W2, b2, epsilon, gamma, beta):
    raise NotImplementedError("write your pallas kernel here")


# speedup vs baseline: 2.2415x; 2.2415x over previous
import functools

import jax
import jax.numpy as jnp
from jax import lax
from jax.experimental import pallas as pl
from jax.experimental.pallas import tpu as pltpu
from jax.experimental.pallas import tpu_sc as plsc

N = 10000
HID = 128
E = 160000
NC = 2
NS = 16
NW = NC * NS
CHUNK = 128
NCHUNK = 40
EPAD = NW * NCHUNK * CHUNK
RPT = 632
NPAD = NS * RPT

_INV_SQRT2 = 0.7071067811865476

def _seg_sum_body(table_hbm, src_hbm, dst_hbm, zeros_hbm, out_hbm,
                  src_v, dst_v, rows_v, acc_sh, sem):
    c = lax.axis_index("c")
    s = lax.axis_index("s")
    wid = c * NS + s
    pltpu.sync_copy(zeros_hbm.at[pl.ds(s * RPT, RPT)],
                    acc_sh.at[pl.ds(s * RPT, RPT)])
    pltpu.sync_copy(src_hbm.at[wid], src_v)
    pltpu.sync_copy(dst_hbm.at[wid], dst_v)
    plsc.subcore_barrier()

    def body(j, carry):
        pltpu.async_copy(table_hbm.at[src_v.at[j]], rows_v, sem).wait()
        pltpu.sync_copy(rows_v, acc_sh.at[dst_v.at[j]], add=True)
        return carry

    lax.fori_loop(0, NCHUNK, body, 0)
    plsc.subcore_barrier()
    pltpu.sync_copy(acc_sh.at[pl.ds(s * RPT, RPT)],
                    out_hbm.at[c].at[pl.ds(s * RPT, RPT)])


@functools.cache
def _make_seg_sum():
    mesh = plsc.VectorSubcoreMesh(
        core_axis_name="c", subcore_axis_name="s",
        num_cores=NC, num_subcores=NS)
    return pl.kernel(
        _seg_sum_body,
        out_type=jax.ShapeDtypeStruct((NC, NPAD, HID), jnp.float32),
        mesh=mesh,
        scratch_types=[
            pltpu.VMEM((NCHUNK, CHUNK), jnp.int32),
            pltpu.VMEM((NCHUNK, CHUNK), jnp.int32),
            pltpu.VMEM((CHUNK, HID), jnp.float32),
            pltpu.VMEM_SHARED((NPAD, HID), jnp.float32),
            pltpu.SemaphoreType.DMA,
        ],
    )


def _mm_relu_body(p_ref, w_ref, b_ref, o_ref):
    agg = p_ref[0, :N, :] + p_ref[1, :N, :]
    o_ref[...] = jnp.maximum(
        jnp.dot(agg, w_ref[...], preferred_element_type=jnp.float32)
        + b_ref[...], 0.0)


_mm_relu = pl.pallas_call(
    _mm_relu_body,
    out_shape=jax.ShapeDtypeStruct((N, HID), jnp.float32),
)


def _combine_body(p_ref, x_ref, w2_ref, b2_ref, wfc_ref, bfc_ref,
                  eps_ref, gamma_ref, beta_ref, g_ref, hn_ref):
    agg2 = p_ref[0, :N, :] + p_ref[1, :N, :]
    h2 = jnp.dot(agg2, w2_ref[...], preferred_element_type=jnp.float32) + b2_ref[...]
    xv = x_ref[...]
    h1 = jnp.dot(xv, wfc_ref[...], preferred_element_type=jnp.float32) + bfc_ref[...]
    eps = eps_ref[...]
    h = (1.0 - eps) * h1 + eps * h2
    mean = jnp.mean(h, axis=0, keepdims=True)
    cent = h - mean
    var = jnp.mean(cent * cent, axis=0, keepdims=True)
    hn_ref[...] = cent / jnp.sqrt(var + 1e-5) * gamma_ref[...] + beta_ref[...]
    g_ref[:, :HID] = h * _INV_SQRT2
    g_ref[:, HID:] = xv * _INV_SQRT2


_combine = pl.pallas_call(
    _combine_body,
    out_shape=[
        jax.ShapeDtypeStruct((N, 2 * HID), jnp.float32),
        jax.ShapeDtypeStruct((N, HID), jnp.float32),
    ],
)

_BM = 512


def _gram_body(a_ref, b_ref, o_ref):
    o_ref[...] = lax.dot_general(
        a_ref[...], b_ref[...], (((1,), (1,)), ((), ())),
        preferred_element_type=jnp.float32)


_gram = pl.pallas_call(
    _gram_body,
    grid=(pl.cdiv(N, _BM), pl.cdiv(N, _BM)),
    in_specs=[
        pl.BlockSpec((_BM, 2 * HID), lambda i, j: (i, 0)),
        pl.BlockSpec((_BM, 2 * HID), lambda i, j: (j, 0)),
    ],
    out_specs=pl.BlockSpec((_BM, _BM), lambda i, j: (i, j)),
    out_shape=jax.ShapeDtypeStruct((N, N), jnp.float32),
    compiler_params=pltpu.CompilerParams(
        dimension_semantics=("parallel", "parallel")),
)


def kernel(x, edge_index, adj, W_fc, b_fc, W1, b1, W2, b2, epsilon, gamma, beta):
    src = edge_index[0]
    dst = edge_index[1]
    pad_src = jnp.full((EPAD - E,), N, dtype=jnp.int32)
    pad_dst = jnp.zeros((EPAD - E,), dtype=jnp.int32)
    src_r = jnp.concatenate([src, pad_src]).reshape(NW, NCHUNK, CHUNK)
    dst_r = jnp.concatenate([dst, pad_dst]).reshape(NW, NCHUNK, CHUNK)
    zrow = jnp.zeros((1, HID), dtype=jnp.float32)
    zeros_nh = jnp.zeros((NPAD, HID), dtype=jnp.float32)

    seg_sum = _make_seg_sum()
    table1 = jnp.concatenate([x, zrow], axis=0)
    p1 = seg_sum(table1, src_r, dst_r, zeros_nh)
    h2a = _mm_relu(p1, W1, b1.reshape(1, HID))
    table2 = jnp.concatenate([h2a, zrow], axis=0)
    p2 = seg_sum(table2, src_r, dst_r, zeros_nh)
    g, hn = _combine(p2, x, W2, b2.reshape(1, HID), W_fc, b_fc.reshape(1, HID),
                     epsilon.reshape(N, 1), gamma.reshape(1, HID),
                     beta.reshape(1, HID))
    ret = _gram(g, g)
    return (ret, hn)


# speedup vs baseline: 2.4393x; 1.0883x over previous
import functools

import jax
import jax.numpy as jnp
from jax import lax
from jax.experimental import pallas as pl
from jax.experimental.pallas import tpu as pltpu
from jax.experimental.pallas import tpu_sc as plsc

N = 10000
HID = 128
E = 160000
NC = 2
NS = 16
NW = NC * NS
CHUNK = 128
NCHUNK = 40
EPAD = NW * NCHUNK * CHUNK
RPT = 632
NPAD = NS * RPT

_INV_SQRT2 = 0.7071067811865476

DEPTH = 2


def _seg_sum_body(table_hbm, src_hbm, dst_hbm, zeros_hbm, out_hbm,
                  src_v, dst_v, rows_v, acc_sh, sem0, sem1):
    sems = (sem0, sem1)
    c = lax.axis_index("c")
    s = lax.axis_index("s")
    wid = c * NS + s
    pltpu.sync_copy(zeros_hbm.at[pl.ds(s * RPT, RPT)],
                    acc_sh.at[pl.ds(s * RPT, RPT)])
    pltpu.sync_copy(src_hbm.at[wid], src_v)
    pltpu.sync_copy(dst_hbm.at[wid], dst_v)
    plsc.subcore_barrier()

    for b in range(DEPTH):
        pltpu.async_copy(table_hbm.at[src_v.at[b]], rows_v.at[b], sems[b])

    def body(jj, carry):
        for b in range(DEPTH):
            j = jj * DEPTH + b
            pltpu.make_async_copy(
                table_hbm.at[src_v.at[j]], rows_v.at[b], sems[b]).wait()
            pltpu.sync_copy(rows_v.at[b], acc_sh.at[dst_v.at[j]], add=True)
            nxt = j + DEPTH

            @pl.when(nxt < NCHUNK)
            def _():
                pltpu.async_copy(
                    table_hbm.at[src_v.at[nxt]], rows_v.at[b], sems[b])
        return carry

    lax.fori_loop(0, NCHUNK // DEPTH, body, 0)
    plsc.subcore_barrier()
    pltpu.sync_copy(acc_sh.at[pl.ds(s * RPT, RPT)],
                    out_hbm.at[c].at[pl.ds(s * RPT, RPT)])


@functools.cache
def _make_seg_sum():
    mesh = plsc.VectorSubcoreMesh(
        core_axis_name="c", subcore_axis_name="s",
        num_cores=NC, num_subcores=NS)
    return pl.kernel(
        _seg_sum_body,
        out_type=jax.ShapeDtypeStruct((NC, NPAD, HID), jnp.float32),
        mesh=mesh,
        scratch_types=[
            pltpu.VMEM((NCHUNK, CHUNK), jnp.int32),
            pltpu.VMEM((NCHUNK, CHUNK), jnp.int32),
            pltpu.VMEM((DEPTH, CHUNK, HID), jnp.float32),
            pltpu.VMEM_SHARED((NPAD, HID), jnp.float32),
            pltpu.SemaphoreType.DMA,
            pltpu.SemaphoreType.DMA,
        ],
    )


def _mm_relu_body(p_ref, w_ref, b_ref, o_ref):
    agg = p_ref[0, :N, :] + p_ref[1, :N, :]
    o_ref[...] = jnp.maximum(
        jnp.dot(agg, w_ref[...], preferred_element_type=jnp.float32)
        + b_ref[...], 0.0)


_mm_relu = pl.pallas_call(
    _mm_relu_body,
    out_shape=jax.ShapeDtypeStruct((N, HID), jnp.float32),
)


def _combine_body(p_ref, x_ref, w2_ref, b2_ref, wfc_ref, bfc_ref,
                  eps_ref, gamma_ref, beta_ref, g_ref, hn_ref):
    agg2 = p_ref[0, :N, :] + p_ref[1, :N, :]
    h2 = jnp.dot(agg2, w2_ref[...], preferred_element_type=jnp.float32) + b2_ref[...]
    xv = x_ref[...]
    h1 = jnp.dot(xv, wfc_ref[...], preferred_element_type=jnp.float32) + bfc_ref[...]
    eps = eps_ref[...]
    h = (1.0 - eps) * h1 + eps * h2
    mean = jnp.mean(h, axis=0, keepdims=True)
    cent = h - mean
    var = jnp.mean(cent * cent, axis=0, keepdims=True)
    hn_ref[...] = cent / jnp.sqrt(var + 1e-5) * gamma_ref[...] + beta_ref[...]
    g_ref[:, :HID] = h * _INV_SQRT2
    g_ref[:, HID:] = xv * _INV_SQRT2


_combine = pl.pallas_call(
    _combine_body,
    out_shape=[
        jax.ShapeDtypeStruct((N, 2 * HID), jnp.float32),
        jax.ShapeDtypeStruct((N, HID), jnp.float32),
    ],
)

_BM = 512


def _gram_body(a_ref, b_ref, o_ref):
    o_ref[...] = lax.dot_general(
        a_ref[...], b_ref[...], (((1,), (1,)), ((), ())),
        preferred_element_type=jnp.float32)


_gram = pl.pallas_call(
    _gram_body,
    grid=(pl.cdiv(N, _BM), pl.cdiv(N, _BM)),
    in_specs=[
        pl.BlockSpec((_BM, 2 * HID), lambda i, j: (i, 0)),
        pl.BlockSpec((_BM, 2 * HID), lambda i, j: (j, 0)),
    ],
    out_specs=pl.BlockSpec((_BM, _BM), lambda i, j: (i, j)),
    out_shape=jax.ShapeDtypeStruct((N, N), jnp.float32),
    compiler_params=pltpu.CompilerParams(
        dimension_semantics=("parallel", "parallel")),
)


def kernel(x, edge_index, adj, W_fc, b_fc, W1, b1, W2, b2, epsilon, gamma, beta):
    src = edge_index[0]
    dst = edge_index[1]
    pad_src = jnp.full((EPAD - E,), N, dtype=jnp.int32)
    pad_dst = jnp.zeros((EPAD - E,), dtype=jnp.int32)
    src_r = jnp.concatenate([src, pad_src]).reshape(NW, NCHUNK, CHUNK)
    dst_r = jnp.concatenate([dst, pad_dst]).reshape(NW, NCHUNK, CHUNK)
    zrow = jnp.zeros((1, HID), dtype=jnp.float32)
    zeros_nh = jnp.zeros((NPAD, HID), dtype=jnp.float32)

    seg_sum = _make_seg_sum()
    table1 = jnp.concatenate([x, zrow], axis=0)
    p1 = seg_sum(table1, src_r, dst_r, zeros_nh)
    h2a = _mm_relu(p1, W1, b1.reshape(1, HID))
    table2 = jnp.concatenate([h2a, zrow], axis=0)
    p2 = seg_sum(table2, src_r, dst_r, zeros_nh)
    g, hn = _combine(p2, x, W2, b2.reshape(1, HID), W_fc, b_fc.reshape(1, HID),
                     epsilon.reshape(N, 1), gamma.reshape(1, HID),
                     beta.reshape(1, HID))
    ret = _gram(g, g)
    return (ret, hn)
